# Initial kernel scaffold; baseline (speedup 1.0000x reference)
import jax
import jax.numpy as jnp
from jax.experimental import pallas as pl


def kernel(x, edge_index, W_enc0, a_src0, a_dst0, b_enc0, W_enc1, b_enc1, W_enc2, b_enc2, W_mu, b_mu, W_lv, b_lv, W_lat, b_lat, W_dec2, b_dec2, W_dec1, b_dec1, W_dec0, b_dec0):
    raise NotImplementedError("write your pallas kernel here")


# speedup vs baseline: 1.7656x; 1.7656x over previous
import functools
import math

import jax
import jax.numpy as jnp
from jax import lax
from jax.experimental import pallas as pl
from jax.experimental.pallas import tpu as pltpu

N = 50000
E = 800000
HID = 128
LAT = 32
CG = 3
ROWS_BLK = 2000


def _row_grid(n_rows):
    return (n_rows // ROWS_BLK,)


def _rb(shape_tail):
    return pl.BlockSpec((ROWS_BLK,) + shape_tail, lambda i: (i,) + (0,) * len(shape_tail))


def _full(shape):
    return pl.BlockSpec(shape, lambda i: (0,) * len(shape))


def _stageA_body(x_ref, w_ref, asr_ref, adr_ref, h_ref, sa_ref, sd_ref, ws_ref):
    h = jnp.dot(x_ref[...], w_ref[...], preferred_element_type=jnp.float32)
    h_ref[...] = h
    sa = jnp.dot(h, asr_ref[...], preferred_element_type=jnp.float32)
    sd = jnp.dot(h, adr_ref[...], preferred_element_type=jnp.float32)
    sa_ref[...] = sa
    sd_ref[...] = sd
    s = sa + sd
    e = jnp.where(s >= 0, s, 0.2 * s)
    ws_ref[...] = jnp.exp(e)


def _stageA(x, W0, a_s, a_d):
    return pl.pallas_call(
        _stageA_body,
        grid=_row_grid(N),
        in_specs=[_rb((3,)), _full((3, HID)), _full((HID, 1)), _full((HID, 1))],
        out_specs=[_rb((HID,)), _rb((1,)), _rb((1,)), _rb((1,))],
        out_shape=[
            jax.ShapeDtypeStruct((N, HID), jnp.float32),
            jax.ShapeDtypeStruct((N, 1), jnp.float32),
            jax.ShapeDtypeStruct((N, 1), jnp.float32),
            jax.ShapeDtypeStruct((N, 1), jnp.float32),
        ],
    )(x, W0, a_s.reshape(HID, 1), a_d.reshape(HID, 1))


def _stageB_body(aggw_ref, den_ref, ws_ref, h0_ref, b0_ref, w1_ref, b1_ref, z1_ref, h1_ref):
    ws = ws_ref[...]
    denom = den_ref[...] + ws + 1e-16
    z1 = (aggw_ref[...] + ws * h0_ref[...]) / denom + b0_ref[...]
    z1_ref[...] = z1
    h1_ref[...] = jnp.dot(z1, w1_ref[...], preferred_element_type=jnp.float32) + b1_ref[...]


def _stageB(aggw, denom_e, wself, h0, b0, W1, b1):
    return pl.pallas_call(
        _stageB_body,
        grid=_row_grid(N),
        in_specs=[_rb((HID,)), _rb((1,)), _rb((1,)), _rb((HID,)), _full((1, HID)),
                  _full((HID, HID)), _full((1, HID))],
        out_specs=[_rb((HID,)), _rb((HID,))],
        out_shape=[
            jax.ShapeDtypeStruct((N, HID), jnp.float32),
            jax.ShapeDtypeStruct((N, HID), jnp.float32),
        ],
    )(aggw, denom_e, wself, h0, b0.reshape(1, HID), W1, b1.reshape(1, HID))


def _stageC_body(agg_ref, cnt_ref, h_ref, z_ref, w_ref, b_ref, z2_ref, h2_ref):
    mean = (agg_ref[...] + h_ref[...]) / (cnt_ref[...] + 1.0)
    z2 = z_ref[...] + jnp.maximum(mean, 0.0)
    z2_ref[...] = z2
    h2_ref[...] = jnp.dot(z2, w_ref[...], preferred_element_type=jnp.float32) + b_ref[...]


def _stageC(agg, cnt, h, z, W, b):
    return pl.pallas_call(
        _stageC_body,
        grid=_row_grid(N),
        in_specs=[_rb((HID,)), _rb((1,)), _rb((HID,)), _rb((HID,)),
                  _full((HID, HID)), _full((1, HID))],
        out_specs=[_rb((HID,)), _rb((HID,))],
        out_shape=[
            jax.ShapeDtypeStruct((N, HID), jnp.float32),
            jax.ShapeDtypeStruct((N, HID), jnp.float32),
        ],
    )(agg, cnt, h, z, W, b.reshape(1, HID))


def _stageD_body(agg_ref, cnt_ref, h_ref, z_ref, wmu_ref, bmu_ref, wlv_ref, blv_ref,
                 eps_ref, wc_ref, bc_ref, out_ref, mu_ref, lv_ref):
    mean = (agg_ref[...] + h_ref[...]) / (cnt_ref[...] + 1.0)
    z3 = z_ref[...] + jnp.maximum(mean, 0.0)
    mu = jnp.dot(z3, wmu_ref[...], preferred_element_type=jnp.float32) + bmu_ref[...]
    lv = jnp.dot(z3, wlv_ref[...], preferred_element_type=jnp.float32) + blv_ref[...]
    mu_ref[...] = mu
    lv_ref[...] = lv
    zr = mu + eps_ref[...] * jnp.exp(0.5 * lv)
    out_ref[...] = jnp.dot(zr, wc_ref[...], preferred_element_type=jnp.float32) + bc_ref[...]


def _stageD(agg, cnt, h, z, W_mu, b_mu, W_lv, b_lv, eps, Wc, bc):
    return pl.pallas_call(
        _stageD_body,
        grid=_row_grid(N),
        in_specs=[_rb((HID,)), _rb((1,)), _rb((HID,)), _rb((HID,)),
                  _full((HID, LAT)), _full((1, LAT)), _full((HID, LAT)), _full((1, LAT)),
                  _rb((LAT,)), _full((LAT, CG)), _full((1, CG))],
        out_specs=[_rb((CG,)), _rb((LAT,)), _rb((LAT,))],
        out_shape=[
            jax.ShapeDtypeStruct((N, CG), jnp.float32),
            jax.ShapeDtypeStruct((N, LAT), jnp.float32),
            jax.ShapeDtypeStruct((N, LAT), jnp.float32),
        ],
    )(agg, cnt, h, z, W_mu, b_mu.reshape(1, LAT), W_lv, b_lv.reshape(1, LAT),
      eps, Wc, bc.reshape(1, CG))


def _edge_aggregate(src, dst, h, w_e=None):
    rows = h[src]
    if w_e is not None:
        rows = rows * w_e[:, None]
    return jax.ops.segment_sum(rows, dst, num_segments=N)


def kernel(x, edge_index, W_enc0, a_src0, a_dst0, b_enc0, W_enc1, b_enc1, W_enc2, b_enc2,
           W_mu, b_mu, W_lv, b_lv, W_lat, b_lat, W_dec2, b_dec2, W_dec1, b_dec1,
           W_dec0, b_dec0):
    src = edge_index[0]
    dst = edge_index[1]

    h0, sa, sd, wself = _stageA(x, W_enc0, a_src0, a_dst0)
    sa1 = sa.reshape(N)
    sd1 = sd.reshape(N)

    s_e = sa1[src] + sd1[dst]
    w_e = jnp.exp(jnp.where(s_e >= 0, s_e, 0.2 * s_e))
    aggw = _edge_aggregate(src, dst, h0, w_e)
    denom_e = jax.ops.segment_sum(w_e, dst, num_segments=N).reshape(N, 1)
    cnt = jax.ops.segment_sum(jnp.ones((E,), jnp.float32), dst, num_segments=N).reshape(N, 1)

    z1, h1 = _stageB(aggw, denom_e, wself, h0, b_enc0, W_enc1, b_enc1)
    agg1 = _edge_aggregate(src, dst, h1)
    z2, h2 = _stageC(agg1, cnt, h1, z1, W_enc2, b_enc2)
    agg2 = _edge_aggregate(src, dst, h2)

    eps = jax.random.normal(jax.random.key(42), (N, LAT), dtype=jnp.float32)
    Wc = W_lat @ W_dec2 @ W_dec1 @ W_dec0
    bc = ((b_lat @ W_dec2 + b_dec2) @ W_dec1 + b_dec1) @ W_dec0 + b_dec0
    out, mu, lv = _stageD(agg2, cnt, h2, z2, W_mu, b_mu, W_lv, b_lv, eps, Wc, bc)
    return (out, mu, lv)


# speedup vs baseline: 5.2970x; 3.0001x over previous
import functools

import jax
import jax.numpy as jnp
from jax import lax
from jax.experimental import pallas as pl
from jax.experimental.pallas import tpu as pltpu
from jax.experimental.pallas import tpu_sc as plsc

N = 50000
E = 800000
HID = 128
LAT = 32
CG = 3

NQ = 16
QN = 3125
QP = 3136
PN = NQ * QP
RPT = 196
ZB = 56
ROWS_BLK = 1568

NS = 16
EPT = E // NS
CH = 2000
NCHUNK = EPT // CH
G = 128
MBUF = CH + G + 16


def _row_grid():
    return (PN // ROWS_BLK,)


def _rb(shape_tail):
    return pl.BlockSpec((ROWS_BLK,) + shape_tail, lambda i: (i,) + (0,) * len(shape_tail))


def _full(shape):
    return pl.BlockSpec(shape, lambda i: (0,) * len(shape))


def _stageA_body(x_ref, w_ref, asr_ref, adr_ref, h_ref, sa_ref, sd_ref, ws_ref):
    h = jnp.dot(x_ref[...], w_ref[...], preferred_element_type=jnp.float32)
    h_ref[...] = h
    sa = jnp.dot(h, asr_ref[...], preferred_element_type=jnp.float32)
    sd = jnp.dot(h, adr_ref[...], preferred_element_type=jnp.float32)
    sa_ref[...] = sa
    sd_ref[...] = sd
    s = sa + sd
    e = jnp.where(s >= 0, s, 0.2 * s)
    ws_ref[...] = jnp.exp(e)


def _stageA(x, W0, a_s, a_d):
    return pl.pallas_call(
        _stageA_body,
        grid=_row_grid(),
        in_specs=[_rb((3,)), _full((3, HID)), _full((HID, 1)), _full((HID, 1))],
        out_specs=[_rb((HID,)), _rb((1,)), _rb((1,)), _rb((1,))],
        out_shape=[
            jax.ShapeDtypeStruct((PN, HID), jnp.float32),
            jax.ShapeDtypeStruct((PN, 1), jnp.float32),
            jax.ShapeDtypeStruct((PN, 1), jnp.float32),
            jax.ShapeDtypeStruct((PN, 1), jnp.float32),
        ],
    )(x, W0, a_s.reshape(HID, 1), a_d.reshape(HID, 1))


def _stageB_body(aggw_ref, den_ref, ws_ref, h0_ref, b0_ref, w1_ref, b1_ref, z1_ref, h1_ref):
    ws = ws_ref[...]
    denom = den_ref[...] + ws + 1e-16
    z1 = (aggw_ref[...] + ws * h0_ref[...]) / denom + b0_ref[...]
    z1_ref[...] = z1
    h1_ref[...] = jnp.dot(z1, w1_ref[...], preferred_element_type=jnp.float32) + b1_ref[...]


def _stageB(aggw, denom_e, wself, h0, b0, W1, b1):
    return pl.pallas_call(
        _stageB_body,
        grid=_row_grid(),
        in_specs=[_rb((HID,)), _rb((1,)), _rb((1,)), _rb((HID,)), _full((1, HID)),
                  _full((HID, HID)), _full((1, HID))],
        out_specs=[_rb((HID,)), _rb((HID,))],
        out_shape=[
            jax.ShapeDtypeStruct((PN, HID), jnp.float32),
            jax.ShapeDtypeStruct((PN, HID), jnp.float32),
        ],
    )(aggw, denom_e, wself, h0, b0.reshape(1, HID), W1, b1.reshape(1, HID))


def _stageC_body(agg_ref, cnt_ref, h_ref, z_ref, w_ref, b_ref, z2_ref, h2_ref):
    mean = (agg_ref[...] + h_ref[...]) / (cnt_ref[...] + 1.0)
    z2 = z_ref[...] + jnp.maximum(mean, 0.0)
    z2_ref[...] = z2
    h2_ref[...] = jnp.dot(z2, w_ref[...], preferred_element_type=jnp.float32) + b_ref[...]


def _stageC(agg, cnt, h, z, W, b):
    return pl.pallas_call(
        _stageC_body,
        grid=_row_grid(),
        in_specs=[_rb((HID,)), _rb((1,)), _rb((HID,)), _rb((HID,)),
                  _full((HID, HID)), _full((1, HID))],
        out_specs=[_rb((HID,)), _rb((HID,))],
        out_shape=[
            jax.ShapeDtypeStruct((PN, HID), jnp.float32),
            jax.ShapeDtypeStruct((PN, HID), jnp.float32),
        ],
    )(agg, cnt, h, z, W, b.reshape(1, HID))


def _stageD_body(agg_ref, cnt_ref, h_ref, z_ref, wmu_ref, bmu_ref, wlv_ref, blv_ref,
                 eps_ref, wc_ref, bc_ref, out_ref, mu_ref, lv_ref):
    mean = (agg_ref[...] + h_ref[...]) / (cnt_ref[...] + 1.0)
    z3 = z_ref[...] + jnp.maximum(mean, 0.0)
    mu = jnp.dot(z3, wmu_ref[...], preferred_element_type=jnp.float32) + bmu_ref[...]
    lv = jnp.dot(z3, wlv_ref[...], preferred_element_type=jnp.float32) + blv_ref[...]
    mu_ref[...] = mu
    lv_ref[...] = lv
    zr = mu + eps_ref[...] * jnp.exp(0.5 * lv)
    out_ref[...] = jnp.dot(zr, wc_ref[...], preferred_element_type=jnp.float32) + bc_ref[...]


def _stageD(agg, cnt, h, z, W_mu, b_mu, W_lv, b_lv, eps, Wc, bc):
    return pl.pallas_call(
        _stageD_body,
        grid=_row_grid(),
        in_specs=[_rb((HID,)), _rb((1,)), _rb((HID,)), _rb((HID,)),
                  _full((HID, LAT)), _full((1, LAT)), _full((HID, LAT)), _full((1, LAT)),
                  _rb((LAT,)), _full((LAT, CG)), _full((1, CG))],
        out_specs=[_rb((CG,)), _rb((LAT,)), _rb((LAT,))],
        out_shape=[
            jax.ShapeDtypeStruct((PN, CG), jnp.float32),
            jax.ShapeDtypeStruct((PN, LAT), jnp.float32),
            jax.ShapeDtypeStruct((PN, LAT), jnp.float32),
        ],
    )(agg, cnt, h, z, W_mu, b_mu.reshape(1, LAT), W_lv, b_lv.reshape(1, LAT),
      eps, Wc, bc.reshape(1, CG))


def _sc_body(mode, refs):
    gat = mode == "gat"
    if gat:
        (hp, srci, dsti, sap, sdp, aggo, deno,
         srcv, dstv, msrc, mldst, wv, gidx, ldst, wcur, rows,
         sa_t, sd_t, zbuf, zsc, cb, acc, cacc) = refs
    elif mode == "cnt":
        (hp, srci, dsti, aggo, deno,
         srcv, dstv, msrc, mldst, gidx, ldst, wcur, rows,
         zbuf, zsc, cb, acc, cacc) = refs
    else:
        (hp, srci, dsti, aggo,
         srcv, dstv, msrc, mldst, gidx, ldst, rows,
         zbuf, zsc, acc) = refs
        cacc = deno = wcur = cb = None

    c = lax.axis_index("c")
    s = lax.axis_index("s")
    i16 = jnp.int32
    lane = lax.iota(i16, 16)
    z16 = jnp.zeros((16,), jnp.float32)
    o16 = jnp.ones((16,), jnp.float32)
    ebase = s * EPT

    def _zrow(i, _):
        for cc in range(8):
            zbuf[i, pl.ds(cc * 16, 16)] = z16
        return 0
    lax.fori_loop(0, zbuf.shape[0], _zrow, 0)
    def _zsc(i, _):
        zsc[pl.ds(i * 16, 16)] = z16
        return 0
    lax.fori_loop(0, (2 * RPT) // 16, _zsc, 0)
    if mode == "cnt":
        def _ones(i, _):
            wcur[pl.ds(i * 16, 16)] = o16
            return 0
        lax.fori_loop(0, G // 16, _ones, 0)

    if gat:
        pltpu.sync_copy(sap, sa_t)

    for p in range(NQ // 2):
        q = 2 * p + c
        lo = q * QN
        hi = lo + QN
        qb = q * QP
        if gat:
            pltpu.sync_copy(sdp.at[pl.ds(pl.multiple_of(qb, 8), QP)], sd_t)

        @pl.when(s < 8)
        def _():
            for k in range(7):
                o = pl.multiple_of(s * 392 + k * ZB, 8)
                pltpu.sync_copy(zbuf, acc.at[pl.ds(o, ZB)])
        if cacc is not None:
            @pl.when(s < 8)
            def _():
                pltpu.sync_copy(zsc, cacc.at[pl.ds(pl.multiple_of(s * (2 * RPT), 8), 2 * RPT)])
        plsc.subcore_barrier()

        def chunk(ch, _):
            eb = pl.multiple_of(ebase + ch * CH, 8)
            pltpu.sync_copy(srci.at[pl.ds(eb, CH)], srcv)
            pltpu.sync_copy(dsti.at[pl.ds(eb, CH)], dstv)

            def comp(v, off):
                d = dstv[pl.ds(v * 16, 16)]
                sv = srcv[pl.ds(v * 16, 16)]
                m = (d >= lo) & (d < hi)
                ld = d - lo
                sp = sv + (QP - QN) * (sv // QN)
                plsc.store_compressed(msrc.at[pl.ds(off, 16)], sp, mask=m)
                plsc.store_compressed(mldst.at[pl.ds(off, 16)], ld, mask=m)
                if gat:
                    sa_v = plsc.load_gather(sa_t, [sp])
                    sd_v = plsc.load_gather(sd_t, [jnp.where(m, ld, 0)])
                    sc_ = sa_v + sd_v
                    e = jnp.where(sc_ >= 0, sc_, 0.2 * sc_)
                    plsc.store_compressed(wv.at[pl.ds(off, 16)], jnp.exp(e), mask=m)
                return off + jnp.sum(m.astype(i16))

            off = lax.fori_loop(0, CH // 16, comp, jnp.int32(0))

            pad_src = lane + s * 16
            for k in range(G // 16):
                msrc[pl.ds(off + k * 16, 16)] = pad_src
                mldst[pl.ds(off + k * 16, 16)] = QN + ((lane + k + s) % (QP - QN))
                if gat:
                    wv[pl.ds(off + k * 16, 16)] = z16
            nb = lax.shift_right_logical(off + (G - 1), 7)

            def sub(b, _):
                for j in range(G // 16):
                    gidx[pl.ds(j * 16, 16)] = msrc[pl.ds(b * G + j * 16, 16)]
                    ldst[pl.ds(j * 16, 16)] = mldst[pl.ds(b * G + j * 16, 16)]
                    if gat:
                        wcur[pl.ds(j * 16, 16)] = wv[pl.ds(b * G + j * 16, 16)]
                pltpu.sync_copy(hp.at[gidx], rows)
                if gat:
                    def scale(i, _):
                        wspl = plsc.load_gather(wcur, [jnp.zeros((16,), i16) + i])
                        for cc in range(8):
                            rows[i, pl.ds(cc * 16, 16)] = rows[i, pl.ds(cc * 16, 16)] * wspl
                        return 0
                    lax.fori_loop(0, G, scale, 0)
                pltpu.sync_copy(rows, acc.at[ldst], add=True)
                if cacc is not None:
                    pltpu.sync_copy(wcur, cacc.at[ldst], add=True)
                return 0

            lax.fori_loop(0, nb, sub, 0)
            return 0

        lax.fori_loop(0, NCHUNK, chunk, 0)
        plsc.subcore_barrier()

        @pl.when(s < 8)
        def _():
            for k in range(7):
                o = pl.multiple_of(s * 392 + k * ZB, 8)
                pltpu.sync_copy(acc.at[pl.ds(o, ZB)],
                                aggo.at[pl.ds(pl.multiple_of(qb, 8) + o, ZB)])
        if deno is not None:
            @pl.when(s < 8)
            def _():
                pltpu.sync_copy(cacc.at[pl.ds(pl.multiple_of(s * (2 * RPT), 8), 2 * RPT)], cb)
                pltpu.sync_copy(cb, deno.at[pl.ds(pl.multiple_of(qb + s * (2 * RPT), 8), 2 * RPT)])
        plsc.subcore_barrier()


def _sc_aggregate(mode, hp, src, dst, sap=None, sdp=None):
    gat = mode == "gat"
    mesh = plsc.VectorSubcoreMesh(core_axis_name="c", subcore_axis_name="s")

    out_type = [jax.ShapeDtypeStruct((PN, HID), jnp.float32)]
    if mode in ("gat", "cnt"):
        out_type.append(jax.ShapeDtypeStruct((PN,), jnp.float32))

    scratch = [
        pltpu.VMEM((CH,), jnp.int32),
        pltpu.VMEM((CH,), jnp.int32),
        pltpu.VMEM((MBUF,), jnp.int32),
        pltpu.VMEM((MBUF,), jnp.int32),
    ]
    if gat:
        scratch.append(pltpu.VMEM((MBUF,), jnp.float32))
    scratch += [
        pltpu.VMEM((G,), jnp.int32),
        pltpu.VMEM((G,), jnp.int32),
    ]
    if mode in ("gat", "cnt"):
        scratch.append(pltpu.VMEM((G,), jnp.float32))
    scratch.append(pltpu.VMEM((G, HID), jnp.float32))
    if gat:
        scratch += [
            pltpu.VMEM((PN,), jnp.float32),
            pltpu.VMEM((QP,), jnp.float32),
        ]
    scratch += [
        pltpu.VMEM((ZB, HID), jnp.float32),
        pltpu.VMEM((2 * RPT,), jnp.float32),
    ]
    if mode in ("gat", "cnt"):
        scratch.append(pltpu.VMEM((2 * RPT,), jnp.float32))
    scratch.append(pltpu.VMEM_SHARED((QP, HID), jnp.float32))
    if mode in ("gat", "cnt"):
        scratch.append(pltpu.VMEM_SHARED((QP,), jnp.float32))

    body = functools.partial(_sc_body, mode)

    def wrapped(*refs):
        body(refs)

    k = pl.kernel(wrapped, out_type=out_type, mesh=mesh, scratch_types=scratch,
                  compiler_params=pltpu.CompilerParams(needs_layout_passes=False))
    if gat:
        return k(hp, src, dst, sap, sdp)
    return k(hp, src, dst)


def _pad_quarters(a):
    d = a.shape[1]
    return jnp.pad(a.reshape(NQ, QN, d), ((0, 0), (0, QP - QN), (0, 0))).reshape(PN, d)


def _unpad_quarters(a):
    return a.reshape(NQ, QP, -1)[:, :QN].reshape(NQ * QN, -1)


def kernel(x, edge_index, W_enc0, a_src0, a_dst0, b_enc0, W_enc1, b_enc1, W_enc2, b_enc2,
           W_mu, b_mu, W_lv, b_lv, W_lat, b_lat, W_dec2, b_dec2, W_dec1, b_dec1,
           W_dec0, b_dec0):
    src = edge_index[0]
    dst = edge_index[1]
    x_p = _pad_quarters(x)

    h0, sa, sd, wself = _stageA(x_p, W_enc0, a_src0, a_dst0)

    aggw, denom_e = _sc_aggregate("gat", h0, src, dst, sa.reshape(PN), sd.reshape(PN))
    z1, h1 = _stageB(aggw, denom_e.reshape(PN, 1), wself, h0, b_enc0, W_enc1, b_enc1)

    agg1, cnt = _sc_aggregate("cnt", h1, src, dst)
    cnt = cnt.reshape(PN, 1)
    z2, h2 = _stageC(agg1, cnt, h1, z1, W_enc2, b_enc2)

    (agg2,) = _sc_aggregate("agg", h2, src, dst)

    eps = jax.random.normal(jax.random.key(42), (N, LAT), dtype=jnp.float32)
    eps_p = _pad_quarters(eps)
    Wc = W_lat @ W_dec2 @ W_dec1 @ W_dec0
    bc = ((b_lat @ W_dec2 + b_dec2) @ W_dec1 + b_dec1) @ W_dec0 + b_dec0
    out_p, mu_p, lv_p = _stageD(agg2, cnt, h2, z2, W_mu, b_mu, W_lv, b_lv, eps_p, Wc, bc)

    out = _unpad_quarters(out_p)
    mu = _unpad_quarters(mu_p)
    lv = _unpad_quarters(lv_p)
    return (out, mu, lv)
